# Initial kernel scaffold; baseline (speedup 1.0000x reference)
import jax
import jax.numpy as jnp
from jax.experimental import pallas as pl


def kernel(img, kernels, offsets_h, offsets_v):
    raise NotImplementedError("write your pallas kernel here")


# speedup vs baseline: 486.7568x; 486.7568x over previous
import jax
import jax.numpy as jnp
from jax.experimental import pallas as pl

_H = 256


def _body(oh, ov, ker, imgb, ph0, ph1, pv0, pv1, out):
    first = pl.program_id(1) == 0

    lint = jax.lax.broadcasted_iota(jnp.int32, (1, _H), 1)
    jlane = lint.astype(jnp.float32)
    jp = (2 * (lint % 128)).astype(jnp.float32)

    nr, nc = imgb.shape[2], imgb.shape[3]
    r_io = jax.lax.broadcasted_iota(jnp.int32, (nr, nc), 0)
    l_io = jax.lax.broadcasted_iota(jnp.int32, (nr, nc), 1)
    diag = {}
    for s in range(-3, 4):
        mask = (l_io - r_io) == s
        for cch in range(3):
            M = imgb[0, cch]
            bd = jnp.sum(jnp.where(mask, M, 0.0), axis=0, keepdims=True)
            for a in range(4):
                b2 = a + s
                if 0 <= b2 <= 3:
                    diag[(a, b2, cch)] = bd[:, b2:b2 + _H]

    def srcw(p_refs, k, t, is_x):
        q = 2 * k + t
        c, kp = q // 9, q % 9
        add = kp // 3 if is_x else kp % 3
        x = (p_refs[c][0, kp] + 1.5) + add
        x = x + (jp + (c + 0.5))
        fl = jnp.floor(x)
        return jnp.where(first, (fl + 1.0) - x, x - fl)

    acc0 = acc1 = acc2 = None
    for k in range(9):
        rk, ck = k // 3, k % 3
        xs = ((oh[0, k] + 1.5) + rk) + (jlane + 0.5)
        ys = ((ov[0, k] + 1.5) + ck) + (jlane + 0.5)
        bx = jnp.floor(xs) - (jlane + (rk + 2))
        by = jnp.floor(ys) - (jlane + (ck + 2))
        w0 = srcw((ph0, ph1), k, 0, True)
        w1 = srcw((ph0, ph1), k, 1, True)
        v0 = srcw((pv0, pv1), k, 0, False)
        v1 = srcw((pv0, pv1), k, 1, False)
        g = []
        for cch in range(3):
            v00 = diag[(rk, ck, cch)]
            v01 = diag[(rk, ck + 1, cch)]
            v10 = diag[(rk + 1, ck, cch)]
            v11 = diag[(rk + 1, ck + 1, cch)]
            g.append((1 - bx) * ((1 - by) * v00 + by * v01)
                     + bx * ((1 - by) * v10 + by * v11))
        g0, g1, g2 = g
        kv = ker[0, k]
        r0 = v0 * (w0 * g0 + w1 * g0) + v1 * (w0 * g1 + w1 * g2)
        r1 = v0 * (w0 * g0 + w1 * g1) + v1 * (w0 * g1 + w1 * g2)
        r2 = v0 * (w0 * g0 + w1 * g1) + v1 * (w0 * g2 + w1 * g2)
        if acc0 is None:
            acc0, acc1, acc2 = kv * r0, kv * r1, kv * r2
        else:
            acc0, acc1, acc2 = acc0 + kv * r0, acc1 + kv * r1, acc2 + kv * r2

    for cch, acc in enumerate((acc0, acc1, acc2)):
        o = acc * 255.0
        out[0, cch] = o - jnp.sin(2 * jnp.pi * o) / (2 * jnp.pi)


def kernel(img, kernels, offsets_h, offsets_v):
    B = img.shape[0]
    imgb = img[:, :, 2:262, 2:266]
    ph = offsets_h.reshape(B, 9, _H, 128, 2)
    pv = offsets_v.reshape(B, 9, _H, 128, 2)
    ph0 = ph[..., 0].reshape(B, 9, 128, _H)
    ph1 = ph[..., 1].reshape(B, 9, 128, _H)
    pv0 = pv[..., 0].reshape(B, 9, 128, _H)
    pv1 = pv[..., 1].reshape(B, 9, 128, _H)

    half = pl.BlockSpec((1, 9, 128, _H), lambda b, h: (b, 0, h, 0))
    full = pl.BlockSpec((1, 9, 128, _H), lambda b, h: (b, 0, 0, 0))
    out = pl.pallas_call(
        _body,
        grid=(B, 2),
        in_specs=[half, half, half,
                  pl.BlockSpec((1, 3, 260, 264), lambda b, h: (b, 0, 0, 0)),
                  full, full, full, full],
        out_specs=pl.BlockSpec((1, 3, 128, _H), lambda b, h: (b, 0, h, 0)),
        out_shape=jax.ShapeDtypeStruct((B, 3, _H, _H), jnp.float32),
    )(offsets_h, offsets_v, kernels, imgb, ph0, ph1, pv0, pv1)
    return jnp.transpose(out, (0, 2, 3, 1))
